# baseline (speedup 1.0000x reference)
import functools

import jax
import jax.numpy as jnp
from jax import lax
from jax.experimental import pallas as pl
from jax.experimental.pallas import tpu as pltpu
from jax.experimental.pallas import tpu_sc as plsc

N_BOXES = 16384
FEATS = 128
TABLE_ROWS = 50
TABLE_STRIDE = 56
GATHER_ROWS = N_BOXES * 4
NUM_WORKERS = 32
ROWS_PER_W = GATHER_ROWS // NUM_WORKERS
BOXES_PER_W = ROWS_PER_W // 4
DMA_ROWS = 128
CHUNK = 128
DMA_PER_CHUNK = CHUNK // DMA_ROWS
N_CHUNK = ROWS_PER_W // CHUNK
NBUF = 6
LANES = 16
VECS = ROWS_PER_W // LANES


def kernel(scale, boxes, row_w, col_w, hei_w, wid_w):
    mesh = plsc.VectorSubcoreMesh(core_axis_name="c", subcore_axis_name="s")

    @functools.partial(
        pl.kernel,
        mesh=mesh,
        out_type=jax.ShapeDtypeStruct((N_BOXES, 4 * FEATS), jnp.float32),
        scratch_types=[
            pltpu.VMEM((ROWS_PER_W,), jnp.float32),
            pltpu.VMEM((LANES,), jnp.float32),
            pltpu.VMEM((ROWS_PER_W // DMA_ROWS, DMA_ROWS), jnp.int32),
            pltpu.VMEM((NBUF, CHUNK, FEATS), jnp.float32),
            pltpu.VMEM_SHARED((4 * TABLE_STRIDE, FEATS), jnp.float32),
            pltpu.SemaphoreType.DMA,
            pltpu.SemaphoreType.DMA,
        ],
    )
    def k(scale_hbm, boxes_hbm, cw_hbm, rw_hbm, ww_hbm, hw_hbm, out_hbm,
          boxes_v, scale_v, idx_v, rows_v, table_s, sem_g, sem_o):
        wid = lax.axis_index("s") * 2 + lax.axis_index("c")
        out_base = pl.multiple_of(wid * BOXES_PER_W, 8)
        tab_cps = []
        @pl.when(lax.axis_index("s") == 0)
        def _():
            for j, t in enumerate((cw_hbm, rw_hbm, ww_hbm, hw_hbm)):
                tab_cps.append(pltpu.async_copy(
                    t, table_s.at[pl.ds(j * TABLE_STRIDE, TABLE_ROWS)],
                    sem_g))
        pltpu.sync_copy(boxes_hbm.at[pl.ds(wid * ROWS_PER_W, ROWS_PER_W)],
                        boxes_v)
        pltpu.sync_copy(scale_hbm, scale_v)
        lane = lax.iota(jnp.int32, LANES)
        s_vec = scale_v[...]
        off = (lane % 4) * TABLE_STRIDE

        @pl.loop(0, VECS // 8)
        def _(r):
            for u in range(8):
                b = boxes_v[pl.ds(r * DMA_ROWS + u * LANES, LANES)]
                idx = (b * s_vec).astype(jnp.int32) + off
                idx_v[r, pl.ds(u * LANES, LANES)] = idx
        @pl.when(lax.axis_index("s") == 0)
        def _():
            for cp in tab_cps:
                cp.wait()
        plsc.subcore_barrier()

        def fire_gathers(c):
            return [pltpu.async_copy(
                table_s.at[idx_v.at[c * DMA_PER_CHUNK + j]],
                rows_v.at[c % NBUF, pl.ds(j * DMA_ROWS, DMA_ROWS)],
                sem_g) for j in range(DMA_PER_CHUNK)]

        out_cps = {}
        g_cps = fire_gathers(0)
        for c in range(N_CHUNK):
            if c + 1 < N_CHUNK:
                if c + 1 >= NBUF:
                    out_cps.pop(c + 1 - NBUF).wait()
                next_g = fire_gathers(c + 1)
            else:
                next_g = None
            for cp in g_cps:
                cp.wait()
            out_cps[c] = pltpu.async_copy(
                rows_v.at[c % NBUF].reshape(CHUNK // 4, 4 * FEATS),
                out_hbm.at[pl.ds(out_base + c * (CHUNK // 4), CHUNK // 4)],
                sem_o)
            g_cps = next_g
        for c in sorted(out_cps):
            out_cps[c].wait()

    scale16 = jnp.tile(scale, LANES // 2)
    return k(scale16, boxes.reshape(-1), col_w, row_w, wid_w, hei_w)


# speedup vs baseline: 1.0381x; 1.0381x over previous
import functools

import jax
import jax.numpy as jnp
from jax import lax
from jax.experimental import pallas as pl
from jax.experimental.pallas import tpu as pltpu
from jax.experimental.pallas import tpu_sc as plsc

N_BOXES = 16384
FEATS = 128
TABLE_ROWS = 50
TABLE_STRIDE = 56
GATHER_ROWS = N_BOXES * 4
NUM_WORKERS = 32
ROWS_PER_W = GATHER_ROWS // NUM_WORKERS
BOXES_PER_W = ROWS_PER_W // 4
DMA_ROWS = 128
CHUNK = 256
DMA_PER_CHUNK = CHUNK // DMA_ROWS
N_CHUNK = ROWS_PER_W // CHUNK
NBUF = 3
IDX_ROWS = GATHER_ROWS // DMA_ROWS
IDX_ROWS_PER_W = IDX_ROWS // NUM_WORKERS


def kernel(scale, boxes, row_w, col_w, hei_w, wid_w):
    mesh = plsc.VectorSubcoreMesh(core_axis_name="c", subcore_axis_name="s")

    @functools.partial(
        pl.kernel,
        mesh=mesh,
        out_type=jax.ShapeDtypeStruct((N_BOXES, 4 * FEATS), jnp.float32),
        scratch_types=[
            pltpu.VMEM((IDX_ROWS_PER_W, DMA_ROWS), jnp.int32),
            pltpu.VMEM((NBUF, CHUNK, FEATS), jnp.float32),
            pltpu.VMEM_SHARED((4 * TABLE_STRIDE, FEATS), jnp.float32),
            pltpu.SemaphoreType.DMA,
            pltpu.SemaphoreType.DMA,
        ],
    )
    def k(idx_hbm, cw_hbm, rw_hbm, ww_hbm, hw_hbm, out_hbm,
          idx_v, rows_v, table_s, sem_g, sem_o):
        wid = lax.axis_index("s") * 2 + lax.axis_index("c")
        out_base = pl.multiple_of(wid * BOXES_PER_W, 8)
        tab_cps = []
        @pl.when(lax.axis_index("s") == 0)
        def _():
            for j, t in enumerate((cw_hbm, rw_hbm, ww_hbm, hw_hbm)):
                tab_cps.append(pltpu.async_copy(
                    t, table_s.at[pl.ds(j * TABLE_STRIDE, TABLE_ROWS)],
                    sem_g))
        pltpu.sync_copy(
            idx_hbm.at[pl.ds(wid * IDX_ROWS_PER_W, IDX_ROWS_PER_W)], idx_v)
        @pl.when(lax.axis_index("s") == 0)
        def _():
            for cp in tab_cps:
                cp.wait()
        plsc.subcore_barrier()

        def fire_gathers(c):
            return [pltpu.async_copy(
                table_s.at[idx_v.at[c * DMA_PER_CHUNK + j]],
                rows_v.at[c % NBUF, pl.ds(j * DMA_ROWS, DMA_ROWS)],
                sem_g) for j in range(DMA_PER_CHUNK)]

        out_cps = {}
        g_cps = fire_gathers(0)
        for c in range(N_CHUNK):
            if c + 1 < N_CHUNK:
                if c + 1 >= NBUF:
                    out_cps.pop(c + 1 - NBUF).wait()
                next_g = fire_gathers(c + 1)
            else:
                next_g = None
            for cp in g_cps:
                cp.wait()
            out_cps[c] = pltpu.async_copy(
                rows_v.at[c % NBUF].reshape(CHUNK // 4, 4 * FEATS),
                out_hbm.at[pl.ds(out_base + c * (CHUNK // 4), CHUNK // 4)],
                sem_o)
            g_cps = next_g
        for c in sorted(out_cps):
            out_cps[c].wait()

    scaled = boxes * jnp.tile(scale, 2)[None, :]
    idx = scaled.astype(jnp.int32) + (jnp.arange(4, dtype=jnp.int32)
                                      * TABLE_STRIDE)[None, :]
    return k(idx.reshape(IDX_ROWS, DMA_ROWS),
             col_w, row_w, wid_w, hei_w)


# speedup vs baseline: 1.0453x; 1.0068x over previous
import functools

import jax
import jax.numpy as jnp
from jax import lax
from jax.experimental import pallas as pl
from jax.experimental.pallas import tpu as pltpu
from jax.experimental.pallas import tpu_sc as plsc

N_BOXES = 16384
FEATS = 128
TABLE_ROWS = 50
TABLE_STRIDE = 56
GATHER_ROWS = N_BOXES * 4
NUM_WORKERS = 32
ROWS_PER_W = GATHER_ROWS // NUM_WORKERS
BOXES_PER_W = ROWS_PER_W // 4
DMA_ROWS = 128
CHUNK = 256
DMA_PER_CHUNK = CHUNK // DMA_ROWS
N_CHUNK = ROWS_PER_W // CHUNK
NBUF = 3
IDX_ROWS = GATHER_ROWS // DMA_ROWS
IDX_ROWS_PER_W = IDX_ROWS // NUM_WORKERS


def kernel(scale, boxes, row_w, col_w, hei_w, wid_w):
    mesh = plsc.VectorSubcoreMesh(core_axis_name="c", subcore_axis_name="s")

    @functools.partial(
        pl.kernel,
        mesh=mesh,
        out_type=jax.ShapeDtypeStruct((N_BOXES, 4 * FEATS), jnp.float32),
        scratch_types=[
            pltpu.VMEM((IDX_ROWS_PER_W, DMA_ROWS), jnp.int32),
            pltpu.VMEM((NBUF, CHUNK, FEATS), jnp.float32),
            pltpu.VMEM_SHARED((4 * TABLE_STRIDE, FEATS), jnp.float32),
            pltpu.SemaphoreType.DMA,
            pltpu.SemaphoreType.DMA,
        ],
    )
    def k(idx_hbm, cw_hbm, rw_hbm, ww_hbm, hw_hbm, out_hbm,
          idx_v, rows_v, table_s, sem_g, sem_o):
        wid = lax.axis_index("s") * 2 + lax.axis_index("c")
        out_base = pl.multiple_of(wid * BOXES_PER_W, 8)
        tab_cps = []
        @pl.when(lax.axis_index("s") == 0)
        def _():
            for j, t in enumerate((cw_hbm, rw_hbm, ww_hbm, hw_hbm)):
                tab_cps.append(pltpu.async_copy(
                    t, table_s.at[pl.ds(j * TABLE_STRIDE, TABLE_ROWS)],
                    sem_g))
        pltpu.sync_copy(
            idx_hbm.at[pl.ds(wid * IDX_ROWS_PER_W, IDX_ROWS_PER_W)], idx_v)
        @pl.when(lax.axis_index("s") == 0)
        def _():
            for cp in tab_cps:
                cp.wait()
        plsc.subcore_barrier()

        def fire_gathers(c):
            return [pltpu.async_copy(
                table_s.at[idx_v.at[c * DMA_PER_CHUNK + j]],
                rows_v.at[c % NBUF, pl.ds(j * DMA_ROWS, DMA_ROWS)],
                sem_g) for j in range(DMA_PER_CHUNK)]

        out_cps = {}
        g_cps = fire_gathers(0)
        for c in range(N_CHUNK):
            if c + 1 < N_CHUNK:
                if c + 1 >= NBUF:
                    out_cps.pop(c + 1 - NBUF).wait()
                next_g = fire_gathers(c + 1)
            else:
                next_g = None
            for cp in g_cps:
                cp.wait()
            out_cps[c] = pltpu.async_copy(
                rows_v.at[c % NBUF].reshape(CHUNK // 4, 4 * FEATS),
                out_hbm.at[pl.ds(out_base + c * (CHUNK // 4), CHUNK // 4)],
                sem_o)
            g_cps = next_g
        for c in sorted(out_cps):
            out_cps[c].wait()

    b128 = boxes.reshape(IDX_ROWS, DMA_ROWS)
    s128 = jnp.tile(scale, DMA_ROWS // 2)[None, :]
    o128 = ((jnp.arange(DMA_ROWS, dtype=jnp.int32) % 4)
            * TABLE_STRIDE)[None, :]
    idx = (b128 * s128).astype(jnp.int32) + o128
    return k(idx, col_w, row_w, wid_w, hei_w)
